# baseline (speedup 1.0000x reference)
import functools

import jax
import jax.numpy as jnp
from jax import lax
from jax.experimental import pallas as pl
from jax.experimental.pallas import tpu as pltpu
from jax.experimental.pallas import tpu_sc as plsc

_N_E = 8192
_E_DIM = 32
_N_TOK = 4096
_T = 256
_TC_ROWS = 2048
_TC_BLKS = _TC_ROWS // _T

_NW = 32
_SC_ROWS = _N_TOK - _TC_ROWS
_ROWS_PW = _SC_ROWS // _NW
_BANDS_PW = _ROWS_PW // 8


def _zz_body(zt_ref, zz_ref):
    zt = zt_ref[...]
    ztsq = zt * zt
    zzt = ztsq[0:1, :]
    for k in range(1, _E_DIM):
        zzt = zzt + ztsq[k:k + 1, :]
    zz_ref[...] = zzt.T


def _zz_pass(z2dt):
    return pl.pallas_call(
        _zz_body,
        out_shape=jax.ShapeDtypeStruct((_N_TOK, 1), jnp.float32),
    )(z2dt)


def _vq_body(z_ref, zz_ref, w_ref, cols_ref, oh_ref, zq_ref, idx_ref,
             idxf_ref, loss_ref, perp_ref, counts, ww_s, ssq):
    t = pl.program_id(0)
    nsteps = pl.num_programs(0)

    z = z_ref[...]
    zz = zz_ref[...]
    w = w_ref[...]
    z2bf = (z + z).astype(jnp.bfloat16)
    wbf = w.astype(jnp.bfloat16)

    @pl.when(t == 0)
    def _init():
        counts[...] = jnp.zeros((1, _N_E), jnp.float32)
        ww_s[...] = jnp.sum(w * w, axis=1)[None, :]
        ssq[0] = 0.0

    mm2 = lax.dot_general(z2bf, wbf, (((1,), (1,)), ((), ())),
                          preferred_element_type=jnp.float32)

    d = (zz + ww_s[...]) - mm2

    dmin = jnp.min(d, axis=1, keepdims=True)
    cols = cols_ref[...]
    idxf = jnp.min(jnp.where(d == dmin, cols, jnp.float32(3e38)),
                   axis=1, keepdims=True)
    idx = idxf.astype(jnp.int32)
    oh = (cols == idxf).astype(jnp.float32)
    ohbf = oh.astype(jnp.bfloat16)

    @pl.when(t < _TC_BLKS)
    def _store_oh():
        oh_ref[...] = oh

    idx_ref[...] = idx
    idxf_ref[...] = idx[:, 0]

    zq = lax.dot_general(ohbf, wbf, (((1,), (0,)), ((), ())),
                         preferred_element_type=jnp.float32)
    err = zq - z
    zq_ref[...] = z + err

    ones_row = jnp.ones((1, _T), jnp.bfloat16)
    counts[...] += lax.dot_general(ones_row, ohbf, (((1,), (0,)), ((), ())),
                                   preferred_element_type=jnp.float32)
    ssq[0] += jnp.sum(err * err)

    @pl.when(t == nsteps - 1)
    def _fini():
        m = ssq[0] * (1.0 / (_N_TOK * _E_DIM))
        loss_ref[0, 0] = m + 0.25 * m
        p = counts[...] * (1.0 / _N_TOK)
        ent = jnp.sum(p * jnp.log(p + 1e-10))
        perp_ref[0, 0] = jnp.exp(-ent)


def _vq_pass(z2d, zz, w, cols):
    grid = _N_TOK // _T
    return pl.pallas_call(
        _vq_body,
        grid=(grid,),
        in_specs=[
            pl.BlockSpec((_T, _E_DIM), lambda t: (t, 0)),
            pl.BlockSpec((_T, 1), lambda t: (t, 0)),
            pl.BlockSpec((_N_E, _E_DIM), lambda t: (0, 0)),
            pl.BlockSpec((1, _N_E), lambda t: (0, 0)),
        ],
        out_specs=[
            pl.BlockSpec((_T, _N_E),
                         lambda t: (jnp.minimum(t, _TC_BLKS - 1), 0)),
            pl.BlockSpec((_T, _E_DIM), lambda t: (t, 0)),
            pl.BlockSpec((_T, 1), lambda t: (t, 0)),
            pl.BlockSpec((_T,), lambda t: (t,)),
            pl.BlockSpec(memory_space=pltpu.SMEM),
            pl.BlockSpec(memory_space=pltpu.SMEM),
        ],
        out_shape=[
            jax.ShapeDtypeStruct((_N_TOK, _N_E), jnp.float32),
            jax.ShapeDtypeStruct((_N_TOK, _E_DIM), jnp.float32),
            jax.ShapeDtypeStruct((_N_TOK, 1), jnp.int32),
            jax.ShapeDtypeStruct((_N_TOK,), jnp.int32),
            jax.ShapeDtypeStruct((1, 1), jnp.float32),
            jax.ShapeDtypeStruct((1, 1), jnp.float32),
        ],
        scratch_shapes=[
            pltpu.VMEM((1, _N_E), jnp.float32),
            pltpu.VMEM((1, _N_E), jnp.float32),
            pltpu.SMEM((1,), jnp.float32),
        ],
    )(z2d, zz, w, cols)


def _sc_fill_onehot(idx_flat, oh_ref_in):
    mesh = plsc.VectorSubcoreMesh(core_axis_name="c", subcore_axis_name="s")

    @functools.partial(
        pl.kernel, mesh=mesh,
        compiler_params=pltpu.CompilerParams(use_tc_tiling_on_sc=True,
                                             needs_layout_passes=False),
        out_type=(),
        scratch_types=[
            pltpu.VMEM((8, _N_E), jnp.float32),
            pltpu.VMEM((_ROWS_PW + 16,), jnp.int32),
        ],
    )
    def k(idx_hbm, oh_hbm, band_v, idx_v):
        wid = lax.axis_index("s") * 2 + lax.axis_index("c")
        base = _TC_ROWS + wid * _ROWS_PW

        pltpu.sync_copy(idx_hbm.at[pl.ds(base, _ROWS_PW)],
                        idx_v.at[pl.ds(0, _ROWS_PW)])

        zv = jnp.zeros((16,), jnp.float32)
        for r in range(8):
            def zbody(i, _, r=r):
                band_v[r, pl.ds(i * 16, 16)] = zv
                return ()
            lax.fori_loop(0, _N_E // 16, zbody, ())

        lanes = lax.iota(jnp.int32, 16)
        msk = lanes < 8
        ones = jnp.ones((16,), jnp.float32)

        def body(g, _):
            cols = idx_v[pl.ds(g * 8, 16)]
            plsc.store_scatter(band_v, [lanes, cols], ones, mask=msk)
            pltpu.sync_copy(band_v, oh_hbm.at[pl.ds(base + g * 8, 8)])
            plsc.store_scatter(band_v, [lanes, cols], zv, mask=msk)
            return ()
        lax.fori_loop(0, _BANDS_PW, body, ())

    k(idx_flat, oh_ref_in)


def kernel(z_list, embedding_w, embedding_z_w):
    z0 = jnp.transpose(z_list[0], (0, 2, 3, 1)).reshape(-1, _E_DIM)
    z1 = jnp.transpose(z_list[1], (0, 2, 3, 1)).reshape(-1, _E_DIM)
    zz0 = _zz_pass(z0.T)
    zz1 = _zz_pass(z1.T)
    cols = jnp.arange(_N_E, dtype=jnp.float32)[None, :]

    oh1p, zq1f, idx1, idxf1, loss1, perp1 = _vq_pass(z1, zz1, embedding_w,
                                                     cols)
    r1 = jax.new_ref(oh1p)
    _sc_fill_onehot(idxf1, r1)
    oh0p, zq0f, idx0, idxf0, loss0, perp0 = _vq_pass(z0, zz0, embedding_z_w,
                                                     cols)
    r0 = jax.new_ref(oh0p)
    _sc_fill_onehot(idxf0, r0)
    oh1 = jax.freeze(r1)
    oh0 = jax.freeze(r0)

    shp = (z_list.shape[1], z_list.shape[3], z_list.shape[4], _E_DIM)
    zq0 = jnp.transpose(zq0f.reshape(shp), (0, 3, 1, 2))
    zq1 = jnp.transpose(zq1f.reshape(shp), (0, 3, 1, 2))

    total_loss = (loss0[0, 0] + loss1[0, 0])
    total_perp = (perp0[0, 0] + perp1[0, 0]) / 2

    return (total_loss, zq0, zq1, total_perp, oh0, oh1, idx0, idx1)


# speedup vs baseline: 1.0171x; 1.0171x over previous
import functools

import jax
import jax.numpy as jnp
from jax import lax
from jax.experimental import pallas as pl
from jax.experimental.pallas import tpu as pltpu
from jax.experimental.pallas import tpu_sc as plsc

_N_E = 8192
_E_DIM = 32
_N_TOK = 4096
_T = 256
_NW = 32


def _vq_body(store_blks, z_ref, w_ref, cols_ref, oh_ref, zq_ref, idx_ref,
             idxf_ref, loss_ref, perp_ref, counts, ww_s, ssq):
    t = pl.program_id(0)
    nsteps = pl.num_programs(0)

    z = z_ref[...]
    w = w_ref[...]
    z2bf = (z + z).astype(jnp.bfloat16)
    wbf = w.astype(jnp.bfloat16)

    @pl.when(t == 0)
    def _init():
        counts[...] = jnp.zeros((1, _N_E), jnp.float32)
        ww_s[...] = jnp.sum(w * w, axis=1)[None, :]
        ssq[0] = 0.0

    mm2 = lax.dot_general(z2bf, wbf, (((1,), (1,)), ((), ())),
                          preferred_element_type=jnp.float32)

    zsq = z * z
    zz = zsq[:, 0:1]
    for k in range(1, _E_DIM):
        zz = zz + zsq[:, k:k + 1]

    d = (zz + ww_s[...]) - mm2

    dmin = jnp.min(d, axis=1, keepdims=True)
    cols = cols_ref[...]
    idxf = jnp.min(jnp.where(d == dmin, cols, jnp.float32(3e38)),
                   axis=1, keepdims=True)
    idx = idxf.astype(jnp.int32)
    oh = (cols == idxf).astype(jnp.float32)
    ohbf = oh.astype(jnp.bfloat16)

    @pl.when(t < store_blks)
    def _store_oh():
        oh_ref[...] = oh

    idx_ref[...] = idx
    idxf_ref[...] = idx[:, 0]

    zq = lax.dot_general(ohbf, wbf, (((1,), (0,)), ((), ())),
                         preferred_element_type=jnp.float32)
    err = zq - z
    zq_ref[...] = z + err

    ones_row = jnp.ones((1, _T), jnp.bfloat16)
    counts[...] += lax.dot_general(ones_row, ohbf, (((1,), (0,)), ((), ())),
                                   preferred_element_type=jnp.float32)
    ssq[0] += jnp.sum(err * err)

    @pl.when(t == nsteps - 1)
    def _fini():
        m = ssq[0] * (1.0 / (_N_TOK * _E_DIM))
        loss_ref[0, 0] = m + 0.25 * m
        p = counts[...] * (1.0 / _N_TOK)
        ent = jnp.sum(p * jnp.log(p + 1e-10))
        perp_ref[0, 0] = jnp.exp(-ent)


def _vq_pass(z2d, w, cols, store_rows):
    grid = _N_TOK // _T
    store_blks = store_rows // _T
    return pl.pallas_call(
        functools.partial(_vq_body, store_blks),
        grid=(grid,),
        in_specs=[
            pl.BlockSpec((_T, _E_DIM), lambda t: (t, 0)),
            pl.BlockSpec((_N_E, _E_DIM), lambda t: (0, 0)),
            pl.BlockSpec((1, _N_E), lambda t: (0, 0)),
        ],
        out_specs=[
            pl.BlockSpec((_T, _N_E),
                         lambda t: (jnp.minimum(t, store_blks - 1), 0)),
            pl.BlockSpec((_T, _E_DIM), lambda t: (t, 0)),
            pl.BlockSpec((_T, 1), lambda t: (t, 0)),
            pl.BlockSpec((_T,), lambda t: (t,)),
            pl.BlockSpec(memory_space=pltpu.SMEM),
            pl.BlockSpec(memory_space=pltpu.SMEM),
        ],
        out_shape=[
            jax.ShapeDtypeStruct((_N_TOK, _N_E), jnp.float32),
            jax.ShapeDtypeStruct((_N_TOK, _E_DIM), jnp.float32),
            jax.ShapeDtypeStruct((_N_TOK, 1), jnp.int32),
            jax.ShapeDtypeStruct((_N_TOK,), jnp.int32),
            jax.ShapeDtypeStruct((1, 1), jnp.float32),
            jax.ShapeDtypeStruct((1, 1), jnp.float32),
        ],
        scratch_shapes=[
            pltpu.VMEM((1, _N_E), jnp.float32),
            pltpu.VMEM((1, _N_E), jnp.float32),
            pltpu.SMEM((1,), jnp.float32),
        ],
    )(z2d, w, cols)


def _sc_fill_onehot(idx_flat, oh_ref_in, base_rows):
    mesh = plsc.VectorSubcoreMesh(core_axis_name="c", subcore_axis_name="s")
    rows_pw = (_N_TOK - base_rows) // _NW
    bands_pw = rows_pw // 8

    @functools.partial(
        pl.kernel, mesh=mesh,
        compiler_params=pltpu.CompilerParams(use_tc_tiling_on_sc=True,
                                             needs_layout_passes=False),
        out_type=(),
        scratch_types=[
            pltpu.VMEM((8, _N_E), jnp.float32),
            pltpu.VMEM((rows_pw + 16,), jnp.int32),
        ],
    )
    def k(idx_hbm, oh_hbm, band_v, idx_v):
        wid = lax.axis_index("s") * 2 + lax.axis_index("c")
        base = base_rows + wid * rows_pw

        pltpu.sync_copy(idx_hbm.at[pl.ds(base, rows_pw)],
                        idx_v.at[pl.ds(0, rows_pw)])

        zv = jnp.zeros((16,), jnp.float32)
        for r in range(8):
            def zbody(i, _, r=r):
                band_v[r, pl.ds(i * 16, 16)] = zv
                return ()
            lax.fori_loop(0, _N_E // 16, zbody, ())

        lanes = lax.iota(jnp.int32, 16)
        msk = lanes < 8
        ones = jnp.ones((16,), jnp.float32)

        def body(g, _):
            cols = idx_v[pl.ds(g * 8, 16)]
            plsc.store_scatter(band_v, [lanes, cols], ones, mask=msk)
            pltpu.sync_copy(band_v, oh_hbm.at[pl.ds(base + g * 8, 8)])
            plsc.store_scatter(band_v, [lanes, cols], zv, mask=msk)
            return ()
        lax.fori_loop(0, bands_pw, body, ())

    k(idx_flat, oh_ref_in)


def kernel(z_list, embedding_w, embedding_z_w):
    z0 = jnp.transpose(z_list[0], (0, 2, 3, 1)).reshape(-1, _E_DIM)
    z1 = jnp.transpose(z_list[1], (0, 2, 3, 1)).reshape(-1, _E_DIM)
    cols = jnp.arange(_N_E, dtype=jnp.float32)[None, :]

    oh1p, zq1f, idx1, idxf1, loss1, perp1 = _vq_pass(z1, embedding_w, cols,
                                                     2048)
    r1 = jax.new_ref(oh1p)
    _sc_fill_onehot(idxf1, r1, 2048)
    oh0p, zq0f, idx0, idxf0, loss0, perp0 = _vq_pass(z0, embedding_z_w, cols,
                                                     3072)
    r0 = jax.new_ref(oh0p)
    _sc_fill_onehot(idxf0, r0, 3072)
    oh1 = jax.freeze(r1)
    oh0 = jax.freeze(r0)

    shp = (z_list.shape[1], z_list.shape[3], z_list.shape[4], _E_DIM)
    zq0 = jnp.transpose(zq0f.reshape(shp), (0, 3, 1, 2))
    zq1 = jnp.transpose(zq1f.reshape(shp), (0, 3, 1, 2))

    total_loss = (loss0[0, 0] + loss1[0, 0])
    total_perp = (perp0[0, 0] + perp1[0, 0]) / 2

    return (total_loss, zq0, zq1, total_perp, oh0, oh1, idx0, idx1)


# speedup vs baseline: 1.1682x; 1.1485x over previous
import functools

import jax
import jax.numpy as jnp
from jax import lax
from jax.experimental import pallas as pl
from jax.experimental.pallas import tpu as pltpu
from jax.experimental.pallas import tpu_sc as plsc

_N_E = 8192
_E_DIM = 32
_N_TOK = 4096
_T = 256

_NW = 32
_ROWS_PW = _N_TOK // _NW
_BANDS_PW = _ROWS_PW // 8


def _vq_body(write_oh, *refs):
    if write_oh:
        (z_ref, w_ref, cols_ref, oh_ref, zq_ref, idx_ref, idxf_ref, loss_ref,
         perp_ref, counts, ww_s, ssq) = refs
    else:
        (z_ref, w_ref, cols_ref, zq_ref, idx_ref, idxf_ref, loss_ref,
         perp_ref, counts, ww_s, ssq) = refs
    t = pl.program_id(0)
    nsteps = pl.num_programs(0)

    z = z_ref[...]
    w = w_ref[...]
    z2bf = (z + z).astype(jnp.bfloat16)
    wbf = w.astype(jnp.bfloat16)

    @pl.when(t == 0)
    def _init():
        counts[...] = jnp.zeros((1, _N_E), jnp.float32)
        ww_s[...] = jnp.sum(w * w, axis=1)[None, :]
        ssq[0] = 0.0

    mm2 = lax.dot_general(z2bf, wbf, (((1,), (1,)), ((), ())),
                          preferred_element_type=jnp.float32)

    zsq = z * z
    zz = zsq[:, 0:1]
    for k in range(1, _E_DIM):
        zz = zz + zsq[:, k:k + 1]

    d = (zz + ww_s[...]) - mm2

    dmin = jnp.min(d, axis=1, keepdims=True)
    cols = cols_ref[...]
    idxf = jnp.min(jnp.where(d == dmin, cols, jnp.float32(3e38)),
                   axis=1, keepdims=True)
    idx = idxf.astype(jnp.int32)
    oh = (cols == idxf).astype(jnp.float32)
    ohbf = oh.astype(jnp.bfloat16)

    if write_oh:
        oh_ref[...] = oh
    idx_ref[...] = idx
    idxf_ref[...] = idx[:, 0]

    zq = lax.dot_general(ohbf, wbf, (((1,), (0,)), ((), ())),
                         preferred_element_type=jnp.float32)
    err = zq - z
    zq_ref[...] = z + err

    ones_row = jnp.ones((1, _T), jnp.bfloat16)
    counts[...] += lax.dot_general(ones_row, ohbf, (((1,), (0,)), ((), ())),
                                   preferred_element_type=jnp.float32)
    ssq[0] += jnp.sum(err * err)

    @pl.when(t == nsteps - 1)
    def _fini():
        m = ssq[0] * (1.0 / (_N_TOK * _E_DIM))
        loss_ref[0, 0] = m + 0.25 * m
        p = counts[...] * (1.0 / _N_TOK)
        ent = jnp.sum(p * jnp.log(p + 1e-10))
        perp_ref[0, 0] = jnp.exp(-ent)


def _vq_pass(z2d, w, cols, write_oh):
    grid = _N_TOK // _T
    out_specs = [
        pl.BlockSpec((_T, _E_DIM), lambda t: (t, 0)),
        pl.BlockSpec((_T, 1), lambda t: (t, 0)),
        pl.BlockSpec((_T,), lambda t: (t,)),
        pl.BlockSpec(memory_space=pltpu.SMEM),
        pl.BlockSpec(memory_space=pltpu.SMEM),
    ]
    out_shape = [
        jax.ShapeDtypeStruct((_N_TOK, _E_DIM), jnp.float32),
        jax.ShapeDtypeStruct((_N_TOK, 1), jnp.int32),
        jax.ShapeDtypeStruct((_N_TOK,), jnp.int32),
        jax.ShapeDtypeStruct((1, 1), jnp.float32),
        jax.ShapeDtypeStruct((1, 1), jnp.float32),
    ]
    if write_oh:
        out_specs.insert(0, pl.BlockSpec((_T, _N_E), lambda t: (t, 0)))
        out_shape.insert(0, jax.ShapeDtypeStruct((_N_TOK, _N_E), jnp.float32))
    return pl.pallas_call(
        functools.partial(_vq_body, write_oh),
        grid=(grid,),
        in_specs=[
            pl.BlockSpec((_T, _E_DIM), lambda t: (t, 0)),
            pl.BlockSpec((_N_E, _E_DIM), lambda t: (0, 0)),
            pl.BlockSpec((1, _N_E), lambda t: (0, 0)),
        ],
        out_specs=out_specs,
        out_shape=out_shape,
        scratch_shapes=[
            pltpu.VMEM((1, _N_E), jnp.float32),
            pltpu.VMEM((1, _N_E), jnp.float32),
            pltpu.SMEM((1,), jnp.float32),
        ],
    )(z2d, w, cols)


def _sc_onehot(idx_flat):
    mesh = plsc.VectorSubcoreMesh(core_axis_name="c", subcore_axis_name="s")

    @functools.partial(
        pl.kernel, mesh=mesh,
        compiler_params=pltpu.CompilerParams(use_tc_tiling_on_sc=True,
                                             needs_layout_passes=False),
        out_type=jax.ShapeDtypeStruct((_N_TOK, _N_E), jnp.float32),
        scratch_types=[
            pltpu.VMEM((8, _N_E), jnp.float32),
            pltpu.VMEM((144,), jnp.int32),
        ],
    )
    def k(idx_hbm, oh_hbm, band_v, idx_v):
        wid = lax.axis_index("s") * 2 + lax.axis_index("c")
        base = wid * _ROWS_PW

        pltpu.sync_copy(idx_hbm.at[pl.ds(base, _ROWS_PW)],
                        idx_v.at[pl.ds(0, _ROWS_PW)])

        zv = jnp.zeros((16,), jnp.float32)
        for r in range(8):
            def zbody(i, _, r=r):
                band_v[r, pl.ds(i * 16, 16)] = zv
                return ()
            lax.fori_loop(0, _N_E // 16, zbody, ())

        lanes = lax.iota(jnp.int32, 16)
        msk = lanes < 8
        ones = jnp.ones((16,), jnp.float32)

        def body(g, _):
            cols = idx_v[pl.ds(g * 8, 16)]
            plsc.store_scatter(band_v, [lanes, cols], ones, mask=msk)
            pltpu.sync_copy(band_v, oh_hbm.at[pl.ds(base + g * 8, 8)])
            plsc.store_scatter(band_v, [lanes, cols], zv, mask=msk)
            return ()
        lax.fori_loop(0, _BANDS_PW, body, ())

    return k(idx_flat)


def kernel(z_list, embedding_w, embedding_z_w):
    z0 = jnp.transpose(z_list[0], (0, 2, 3, 1)).reshape(-1, _E_DIM)
    z1 = jnp.transpose(z_list[1], (0, 2, 3, 1)).reshape(-1, _E_DIM)

    cols = jnp.arange(_N_E, dtype=jnp.float32)[None, :]

    zq1f, idx1, idxf1, loss1, perp1 = _vq_pass(z1, embedding_w, cols, False)
    oh1 = _sc_onehot(idxf1)
    oh0, zq0f, idx0, _idxf0, loss0, perp0 = _vq_pass(z0, embedding_z_w, cols,
                                                     True)

    shp = (z_list.shape[1], z_list.shape[3], z_list.shape[4], _E_DIM)
    zq0 = jnp.transpose(zq0f.reshape(shp), (0, 3, 1, 2))
    zq1 = jnp.transpose(zq1f.reshape(shp), (0, 3, 1, 2))

    total_loss = (loss0[0, 0] + loss1[0, 0])
    total_perp = (perp0[0, 0] + perp1[0, 0]) / 2

    return (total_loss, zq0, zq1, total_perp, oh0, oh1, idx0, idx1)
